# baseline (speedup 1.0000x reference)
import functools

import jax
import jax.numpy as jnp
from jax import lax
from jax.experimental import pallas as pl
from jax.experimental.pallas import tpu as pltpu
from jax.experimental.pallas import tpu_sc as plsc

D = 16
NC, NS = 2, 16
NW = NC * NS
TPW = 128 // NW
NJ = 200
STEPS = NJ * TPW


def _make_relayout():
    mesh = plsc.VectorSubcoreMesh(core_axis_name="c", subcore_axis_name="s")
    blocks = 15625
    per_w = -(-blocks // NW)

    @functools.partial(
        pl.kernel,
        mesh=mesh,
        out_type=jax.ShapeDtypeStruct((blocks, 8, 8, D), jnp.float32),
        scratch_types=[pltpu.SemaphoreType.DMA],
        compiler_params=pltpu.CompilerParams(
            use_tc_tiling_on_sc=False, needs_layout_passes=False),
    )
    def r(cb4_hbm, out_hbm, sem):
        wid = lax.axis_index("s") * NC + lax.axis_index("c")
        start = jnp.minimum(wid * per_w, blocks - per_w)
        pltpu.async_copy(cb4_hbm.at[pl.ds(start, per_w)],
                         out_hbm.at[pl.ds(start, per_w)], sem).wait()

    return r


def _make_gather():
    mesh = plsc.VectorSubcoreMesh(core_axis_name="c", subcore_axis_name="s")

    @functools.partial(
        pl.kernel,
        mesh=mesh,
        out_type=jax.ShapeDtypeStruct((NJ, 2, 128, 8, 128), jnp.float32),
        scratch_types=[
            pltpu.VMEM((NJ // 8, TPW, 8, 128), jnp.int32),
            pltpu.VMEM((128, D), jnp.float32),
            pltpu.VMEM((128, D), jnp.float32),
            pltpu.VMEM((2, 8, 129), jnp.float32),
            pltpu.VMEM((2, 8, 129), jnp.float32),
            pltpu.SemaphoreType.DMA,
            pltpu.SemaphoreType.DMA,
            pltpu.SemaphoreType.DMA,
            pltpu.SemaphoreType.DMA,
        ],
        compiler_params=pltpu.CompilerParams(
            use_tc_tiling_on_sc=False, needs_layout_passes=False),
    )
    def k(cb_hbm, idx_hbm, out_hbm, idx_v, rows0, rows1, tr0, tr1,
          sg0, sg1, so0, so1):
        rows = (rows0, rows1)
        trs = (tr0, tr1)
        sgs = (sg0, sg1)
        sos = (so0, so1)
        wid = lax.axis_index("s") * NC + lax.axis_index("c")
        t0 = wid * TPW

        def idx_slice(step):
            jl = step % NJ
            tl = step // NJ
            return idx_v.at[jl // 8, tl, jl % 8]

        pltpu.sync_copy(idx_hbm.at[:, pl.ds(t0, TPW)], idx_v)
        pltpu.async_copy(cb_hbm.at[idx_slice(0)], rows0, sg0)
        pltpu.async_copy(cb_hbm.at[idx_slice(1)], rows1, sg1)

        lanes = lax.iota(jnp.int32, 16)

        def body(outer, carry):
            for b in range(2):
                step = outer * 2 + b
                jl = step % NJ
                tg = t0 + step // NJ
                pltpu.make_async_copy(
                    cb_hbm.at[pl.ds(0, 128)], rows[b], sgs[b]).wait()
                @pl.when(step >= 2)
                def _():
                    pltpu.make_async_copy(
                        trs[b].at[:, :, pl.ds(0, 128)],
                        out_hbm.at[0, :, 0], sos[b]).wait()
                rb, tb = rows[b], trs[b]

                @plsc.parallel_loop(0, 128, 1, unroll=16)
                def _(i):
                    v = rb[i]
                    plsc.store_scatter(
                        tb, [lanes // 8, lanes % 8, jnp.full((16,), i, jnp.int32)], v)
                @pl.when(step + 2 < STEPS)
                def _():
                    pltpu.async_copy(
                        cb_hbm.at[idx_slice(step + 2)], rows[b], sgs[b])
                pltpu.async_copy(trs[b].at[:, :, pl.ds(0, 128)],
                                 out_hbm.at[jl, :, tg], sos[b])
            return carry

        lax.fori_loop(0, STEPS // 2, body, 0)
        pltpu.make_async_copy(
            tr0.at[:, :, pl.ds(0, 128)], out_hbm.at[0, :, 0], so0).wait()
        pltpu.make_async_copy(
            tr1.at[:, :, pl.ds(0, 128)], out_hbm.at[0, :, 0], so1).wait()

    return k


def kernel(data, codebook, indices):
    del data
    idx = indices.astype(jnp.int32)
    idx = (idx & -64) | ((idx & 7) << 3) | ((idx >> 3) & 7)
    idx4 = idx.T.reshape(NJ // 8, 8, 128, 128).transpose(0, 2, 1, 3)
    cb4 = codebook.reshape(15625, 8, 8, D).transpose(0, 2, 1, 3)
    cbl = _make_relayout()(cb4).reshape(1000000, D)
    a = _make_gather()(cbl, idx4)
    return a.transpose(2, 4, 0, 1, 3).reshape(16384, NJ, D)


# speedup vs baseline: 4.2428x; 4.2428x over previous
import functools

import jax
import jax.numpy as jnp
from jax import lax
from jax.experimental import pallas as pl
from jax.experimental.pallas import tpu as pltpu
from jax.experimental.pallas import tpu_sc as plsc

D = 16
NC, NS = 2, 16
NW = NC * NS
TPW = 128 // NW
NJ = 200
STEPS = NJ * TPW


def _make_gather():
    mesh = plsc.VectorSubcoreMesh(core_axis_name="c", subcore_axis_name="s")

    @functools.partial(
        pl.kernel,
        mesh=mesh,
        out_type=jax.ShapeDtypeStruct((NJ, 2, 128, 8, 128), jnp.float32),
        scratch_types=[
            pltpu.VMEM((NJ // 8, TPW, 8, 128), jnp.int32),
            pltpu.VMEM((128, D), jnp.float32),
            pltpu.VMEM((128, D), jnp.float32),
            pltpu.VMEM((128, D), jnp.float32),
            pltpu.VMEM((128, D), jnp.float32),
            pltpu.VMEM((2, 8, 129), jnp.float32),
            pltpu.VMEM((2, 8, 129), jnp.float32),
            pltpu.SemaphoreType.DMA,
            pltpu.SemaphoreType.DMA,
            pltpu.SemaphoreType.DMA,
            pltpu.SemaphoreType.DMA,
            pltpu.SemaphoreType.DMA,
            pltpu.SemaphoreType.DMA,
        ],
        compiler_params=pltpu.CompilerParams(
            use_tc_tiling_on_sc=False, needs_layout_passes=False),
    )
    def k(cb_hbm, idx_hbm, out_hbm, idx_v, rows0, rows1, rows2, rows3,
          tr0, tr1, sg0, sg1, sg2, sg3, so0, so1):
        rows = (rows0, rows1, rows2, rows3)
        trs = (tr0, tr1)
        sgs = (sg0, sg1, sg2, sg3)
        sos = (so0, so1)
        wid = lax.axis_index("s") * NC + lax.axis_index("c")
        t0 = wid * TPW

        def idx_slice(step):
            jl = step % NJ
            tl = step // NJ
            return idx_v.at[jl // 8, tl, jl % 8]

        pltpu.sync_copy(idx_hbm.at[:, pl.ds(t0, TPW)], idx_v)
        pltpu.async_copy(cb_hbm.at[idx_slice(0)], rows0, sg0)
        pltpu.async_copy(cb_hbm.at[idx_slice(1)], rows1, sg1)
        pltpu.async_copy(cb_hbm.at[idx_slice(2)], rows2, sg2)
        pltpu.async_copy(cb_hbm.at[idx_slice(3)], rows3, sg3)

        lanes = lax.iota(jnp.int32, 16)

        def body(outer, carry):
            for b in range(4):
                step = outer * 4 + b
                jl = step % NJ
                tg = t0 + step // NJ
                pltpu.make_async_copy(
                    cb_hbm.at[pl.ds(0, 128)], rows[b], sgs[b]).wait()
                @pl.when(step >= 2)
                def _():
                    pltpu.make_async_copy(
                        trs[b % 2].at[:, :, pl.ds(0, 128)],
                        out_hbm.at[0, :, 0], sos[b % 2]).wait()
                rb, tb = rows[b], trs[b % 2]

                @plsc.parallel_loop(0, 128, 1, unroll=16)
                def _(i):
                    v = rb[i]
                    plsc.store_scatter(
                        tb, [lanes // 8, lanes % 8, jnp.full((16,), i, jnp.int32)], v)
                @pl.when(step + 4 < STEPS)
                def _():
                    pltpu.async_copy(
                        cb_hbm.at[idx_slice(step + 4)], rows[b], sgs[b])
                pltpu.async_copy(trs[b % 2].at[:, :, pl.ds(0, 128)],
                                 out_hbm.at[jl, :, tg], sos[b % 2])
            return carry

        lax.fori_loop(0, STEPS // 4, body, 0)
        pltpu.make_async_copy(
            tr0.at[:, :, pl.ds(0, 128)], out_hbm.at[0, :, 0], so0).wait()
        pltpu.make_async_copy(
            tr1.at[:, :, pl.ds(0, 128)], out_hbm.at[0, :, 0], so1).wait()

    return k


def kernel(data, codebook, indices):
    del data
    idx4 = (indices.astype(jnp.int32).T
            .reshape(NJ // 8, 8, 128, 128).transpose(0, 2, 1, 3))
    a = _make_gather()(codebook, idx4)
    return a.transpose(2, 4, 0, 1, 3).reshape(16384, NJ, D)
